# baseline (speedup 1.0000x reference)
import functools

import jax
import jax.numpy as jnp
from jax import lax
from jax.experimental import pallas as pl
from jax.experimental.pallas import tpu as pltpu
from jax.experimental.pallas import tpu_sc as plsc

V = 100000
D = 64
B = 16384
K = 512

NC = 2
NS = 16
NW = NC * NS
BPW = B // NW
CH = 128
NCH = BPW // CH
KPC = K // NC
NKCH = KPC // CH
PAD = 17
L = 16


def _sc_body(center, pos, neg, in_tab, out_tab,
             scores_hbm, sums_hbm,
             cidx, pidx, nidx, zidx, oidx,
             in_rows, pos_rows, neg_rows,
             part, scores_v, zero2,
             shared_sums,
             sem, nsem):
    cid = lax.axis_index("c")
    sid = lax.axis_index("s")
    wid = sid * NC + cid
    base = wid * BPW

    for j in range(NCH):
        pltpu.sync_copy(center.at[pl.ds(base + j * CH, CH)], cidx.at[j])
        pltpu.sync_copy(pos.at[pl.ds(base + j * CH, CH)], pidx.at[j])

    for j in range(NCH):
        for q in range(CH // L):
            zidx[j, pl.ds(q * L, L)] = jnp.zeros((L,), jnp.int32)
    for j in range(NKCH):
        for q in range(CH // L):
            oidx[j, pl.ds(q * L, L)] = jnp.ones((L,), jnp.int32)
    for q in range(D // L):
        zero2[0, pl.ds(q * L, L)] = jnp.zeros((L,), jnp.float32)
        zero2[1, pl.ds(q * L, L)] = jnp.zeros((L,), jnp.float32)

    copies = []
    for j in range(NCH):
        copies.append(pltpu.async_copy(
            in_tab.at[cidx.at[j]], in_rows.at[pl.ds(j * CH, CH)], sem))
        copies.append(pltpu.async_copy(
            out_tab.at[pidx.at[j]], pos_rows.at[pl.ds(j * CH, CH)], sem))

    @pl.when(sid == 0)
    def _():
        for j in range(NKCH):
            pltpu.sync_copy(neg.at[pl.ds(cid * KPC + j * CH, CH)], nidx.at[j])
        neg_copies = []
        for j in range(NKCH):
            neg_copies.append(pltpu.async_copy(
                out_tab.at[nidx.at[j]], neg_rows.at[pl.ds(j * CH, CH)], nsem))
        pltpu.sync_copy(zero2, shared_sums)
        for c in neg_copies:
            c.wait()
        for j in range(NKCH):
            pltpu.sync_copy(neg_rows.at[pl.ds(j * CH, CH)],
                            shared_sums.at[oidx.at[j]], add=True)

    plsc.subcore_barrier()
    for c in copies:
        c.wait()

    for j in range(NCH):
        pltpu.sync_copy(in_rows.at[pl.ds(j * CH, CH)],
                        shared_sums.at[zidx.at[j]], add=True)

    lanes = lax.iota(jnp.int32, L)
    UNROLL = 4

    def prod_body(i, carry):
        for u in range(UNROLL):
            b = i * UNROLL + u
            acc = in_rows[b, pl.ds(0, L)] * pos_rows[b, pl.ds(0, L)]
            for q in range(1, D // L):
                acc += in_rows[b, pl.ds(q * L, L)] * pos_rows[b, pl.ds(q * L, L)]
            plsc.store_scatter(part, [b * PAD + lanes], acc)
        return carry

    lax.fori_loop(0, BPW // UNROLL, prod_body, 0)

    def red_body(t, carry):
        rowbase = (t * L + lanes) * PAD
        acc = plsc.load_gather(part, [rowbase])
        for jcol in range(1, L):
            acc += plsc.load_gather(part, [rowbase + jcol])
        scores_v[pl.ds(t * L, L)] = acc
        return carry

    lax.fori_loop(0, BPW // L, red_body, 0)

    pltpu.sync_copy(scores_v, scores_hbm.at[pl.ds(base, BPW)])

    plsc.subcore_barrier()

    @pl.when(sid == 0)
    def _():
        pltpu.sync_copy(shared_sums, sums_hbm.at[cid])


@functools.partial(
    pl.kernel,
    out_type=(
        jax.ShapeDtypeStruct((B,), jnp.float32),
        jax.ShapeDtypeStruct((NC, 2, D), jnp.float32),
    ),
    mesh=plsc.VectorSubcoreMesh(core_axis_name="c", subcore_axis_name="s"),
    compiler_params=pltpu.CompilerParams(
        needs_layout_passes=False, use_tc_tiling_on_sc=False),
    scratch_types=[
        pltpu.VMEM((NCH, CH), jnp.int32),
        pltpu.VMEM((NCH, CH), jnp.int32),
        pltpu.VMEM((NKCH, CH), jnp.int32),
        pltpu.VMEM((NCH, CH), jnp.int32),
        pltpu.VMEM((NKCH, CH), jnp.int32),
        pltpu.VMEM((BPW, D), jnp.float32),
        pltpu.VMEM((BPW, D), jnp.float32),
        pltpu.VMEM((KPC, D), jnp.float32),
        pltpu.VMEM((BPW * PAD,), jnp.float32),
        pltpu.VMEM((BPW,), jnp.float32),
        pltpu.VMEM((2, D), jnp.float32),
        pltpu.VMEM_SHARED((2, D), jnp.float32),
        pltpu.SemaphoreType.DMA,
        pltpu.SemaphoreType.DMA,
    ],
)
def _sc_stage(center, pos, neg, in_tab, out_tab, scores_hbm, sums_hbm,
              cidx, pidx, nidx, zidx, oidx, in_rows, pos_rows, neg_rows,
              part, scores_v, zero2, shared_sums, sem, nsem):
    _sc_body(center, pos, neg, in_tab, out_tab, scores_hbm, sums_hbm,
             cidx, pidx, nidx, zidx, oidx, in_rows, pos_rows, neg_rows,
             part, scores_v, zero2, shared_sums, sem, nsem)


def _tc_body(scores_ref, sums_ref, out_ref):
    s = scores_ref[...]
    total = jnp.sum(jax.nn.log_sigmoid(s))
    c = sums_ref[0:1, :] + sums_ref[1:2, :]
    ns = jnp.sum(c[:, 0:D] * c[:, D:2 * D])
    out_ref[...] = jnp.reshape(-(total + B * jax.nn.log_sigmoid(-ns)), (1, 1))


def kernel(center_word, positive_words, negative_words, input_table, output_table):
    scores, sums = _sc_stage(
        center_word.astype(jnp.int32),
        positive_words.astype(jnp.int32),
        negative_words.astype(jnp.int32),
        input_table, output_table)
    out = pl.pallas_call(
        _tc_body,
        out_shape=jax.ShapeDtypeStruct((1, 1), jnp.float32),
    )(scores.reshape(128, 128), sums.reshape(2, 2 * D))
    return out[0, 0]


# speedup vs baseline: 1.0698x; 1.0698x over previous
import functools

import jax
import jax.numpy as jnp
from jax import lax
from jax.experimental import pallas as pl
from jax.experimental.pallas import tpu as pltpu
from jax.experimental.pallas import tpu_sc as plsc

V = 100000
D = 64
DP = 128
B = 16384
K = 512

NC = 2
NS = 16
NW = NC * NS
BPW = B // NW
CH = 128
NCH = BPW // CH
KPC = K // NC
NKCH = KPC // CH
PAD = 17
L = 16
QN = D // L


def _sc_body(center, pos, neg, in_tab, out_tab,
             scores_hbm, sums_hbm,
             cidx, pidx, nidx, zidx, oidx,
             in_buf, pos_buf, neg_buf,
             part, scores_v, zero2,
             shared_sums,
             sem, sem2, nsem):
    cid = lax.axis_index("c")
    sid = lax.axis_index("s")
    wid = sid * NC + cid
    base = wid * BPW

    for j in range(NCH):
        pltpu.sync_copy(center.at[pl.ds(base + j * CH, CH)], cidx.at[j])
        pltpu.sync_copy(pos.at[pl.ds(base + j * CH, CH)], pidx.at[j])

    for q in range(CH // L):
        zidx[0, pl.ds(q * L, L)] = jnp.zeros((L,), jnp.int32)
        oidx[0, pl.ds(q * L, L)] = jnp.ones((L,), jnp.int32)
    for q in range(2 * DP // L):
        zero2[q // (DP // L), pl.ds((q % (DP // L)) * L, L)] = (
            jnp.zeros((L,), jnp.float32))

    @pl.when(sid == 0)
    def _():
        pltpu.sync_copy(zero2, shared_sums)

    copies = {}
    for j in range(NCH):
        copies[j] = None

    def fire(j):
        bsel = j % 2
        s = sem if bsel == 0 else sem2
        a = pltpu.async_copy(in_tab.at[cidx.at[j]], in_buf.at[bsel], s)
        b = pltpu.async_copy(out_tab.at[pidx.at[j]], pos_buf.at[bsel], s)
        return (a, b)

    copies[0] = fire(0)

    plsc.subcore_barrier()

    lanes = lax.iota(jnp.int32, L)
    UNROLL = 4

    for j in range(NCH):
        if j + 1 < NCH:
            copies[j + 1] = fire(j + 1)
        for c in copies[j]:
            c.wait()
        bsel = j % 2

        pltpu.sync_copy(in_buf.at[bsel], shared_sums.at[zidx.at[0]], add=True)

        def prod_body(i, carry, _j=j, _bsel=bsel):
            for u in range(UNROLL):
                b = i * UNROLL + u
                acc = (in_buf[_bsel, b, pl.ds(0, L)] *
                       pos_buf[_bsel, b, pl.ds(0, L)])
                for q in range(1, QN):
                    acc += (in_buf[_bsel, b, pl.ds(q * L, L)] *
                            pos_buf[_bsel, b, pl.ds(q * L, L)])
                plsc.store_scatter(part, [(_j * CH + b) * PAD + lanes], acc)
            return carry

        lax.fori_loop(0, CH // UNROLL, prod_body, 0)

    @pl.when(sid == 0)
    def _():
        for j in range(NKCH):
            pltpu.sync_copy(neg.at[pl.ds(cid * KPC + j * CH, CH)], nidx.at[j])
        nc0 = pltpu.async_copy(out_tab.at[nidx.at[0]], neg_buf.at[0], nsem)
        nc1 = pltpu.async_copy(out_tab.at[nidx.at[1]], neg_buf.at[1], nsem)
        nc0.wait()
        pltpu.sync_copy(neg_buf.at[0], shared_sums.at[oidx.at[0]], add=True)
        nc1.wait()
        pltpu.sync_copy(neg_buf.at[1], shared_sums.at[oidx.at[0]], add=True)

    def red_body(t, carry):
        rowbase = (t * L + lanes) * PAD
        acc = plsc.load_gather(part, [rowbase])
        for jcol in range(1, L):
            acc += plsc.load_gather(part, [rowbase + jcol])
        scores_v[pl.ds(t * L, L)] = acc
        return carry

    lax.fori_loop(0, BPW // L, red_body, 0)

    pltpu.sync_copy(scores_v, scores_hbm.at[pl.ds(base, BPW)])

    plsc.subcore_barrier()

    @pl.when(sid == 0)
    def _():
        pltpu.sync_copy(shared_sums, sums_hbm.at[cid])


@functools.partial(
    pl.kernel,
    out_type=(
        jax.ShapeDtypeStruct((B,), jnp.float32),
        jax.ShapeDtypeStruct((NC, 2, DP), jnp.float32),
    ),
    mesh=plsc.VectorSubcoreMesh(core_axis_name="c", subcore_axis_name="s"),
    compiler_params=pltpu.CompilerParams(
        needs_layout_passes=False, use_tc_tiling_on_sc=False),
    scratch_types=[
        pltpu.VMEM((NCH, CH), jnp.int32),
        pltpu.VMEM((NCH, CH), jnp.int32),
        pltpu.VMEM((NKCH, CH), jnp.int32),
        pltpu.VMEM((1, CH), jnp.int32),
        pltpu.VMEM((1, CH), jnp.int32),
        pltpu.VMEM((2, CH, DP), jnp.float32),
        pltpu.VMEM((2, CH, DP), jnp.float32),
        pltpu.VMEM((2, CH, DP), jnp.float32),
        pltpu.VMEM((BPW * PAD,), jnp.float32),
        pltpu.VMEM((BPW,), jnp.float32),
        pltpu.VMEM((2, DP), jnp.float32),
        pltpu.VMEM_SHARED((2, DP), jnp.float32),
        pltpu.SemaphoreType.DMA,
        pltpu.SemaphoreType.DMA,
        pltpu.SemaphoreType.DMA,
    ],
)
def _sc_stage(center, pos, neg, in_tab, out_tab, scores_hbm, sums_hbm,
              cidx, pidx, nidx, zidx, oidx, in_buf, pos_buf, neg_buf,
              part, scores_v, zero2, shared_sums, sem, sem2, nsem):
    _sc_body(center, pos, neg, in_tab, out_tab, scores_hbm, sums_hbm,
             cidx, pidx, nidx, zidx, oidx, in_buf, pos_buf, neg_buf,
             part, scores_v, zero2, shared_sums, sem, sem2, nsem)


def _tc_body(scores_ref, sums_ref, out_ref):
    s = scores_ref[...]
    total = jnp.sum(jax.nn.log_sigmoid(s))
    sm = sums_ref[...]
    sum_in = sm[0:1, :] + sm[2:3, :]
    sum_neg = sm[1:2, :] + sm[3:4, :]
    ns = jnp.sum(sum_in * sum_neg)
    out_ref[...] = jnp.reshape(-(total + B * jax.nn.log_sigmoid(-ns)), (1, 1))


def kernel(center_word, positive_words, negative_words, input_table, output_table):
    in_tab = jnp.pad(input_table, ((0, 0), (0, DP - D)))
    out_tab = jnp.pad(output_table, ((0, 0), (0, DP - D)))
    scores, sums = _sc_stage(
        center_word.astype(jnp.int32),
        positive_words.astype(jnp.int32),
        negative_words.astype(jnp.int32),
        in_tab, out_tab)
    out = pl.pallas_call(
        _tc_body,
        out_shape=jax.ShapeDtypeStruct((1, 1), jnp.float32),
    )(scores.reshape(128, 128), sums.reshape(2 * 2, DP))
    return out[0, 0]


# speedup vs baseline: 1.1019x; 1.0300x over previous
import functools

import jax
import jax.numpy as jnp
from jax import lax
from jax.experimental import pallas as pl
from jax.experimental.pallas import tpu as pltpu
from jax.experimental.pallas import tpu_sc as plsc

V = 100000
D = 64
DP = 128
B = 16384
K = 512

NC = 2
NS = 16
NW = NC * NS
BPW = B // NW
CH = 128
NCH = BPW // CH
KPC = K // NC
NKCH = KPC // CH
PAD = 17
L = 16
QN = D // L


def _sc_body(center, pos, neg, in_tab, out_tab,
             scores_hbm, sums_hbm,
             cidx, pidx, nidx,
             in_buf, pos_buf, neg_buf,
             part, scores_v, sums_v,
             sem, sem2, nsem):
    cid = lax.axis_index("c")
    sid = lax.axis_index("s")
    wid = sid * NC + cid
    base = wid * BPW

    for j in range(NCH):
        pltpu.sync_copy(center.at[pl.ds(base + j * CH, CH)], cidx.at[j])
        pltpu.sync_copy(pos.at[pl.ds(base + j * CH, CH)], pidx.at[j])

    def fire(j):
        bsel = j % 2
        s = sem if bsel == 0 else sem2
        a = pltpu.async_copy(in_tab.at[cidx.at[j]], in_buf.at[bsel], s)
        b = pltpu.async_copy(out_tab.at[pidx.at[j]], pos_buf.at[bsel], s)
        return (a, b)

    copies = {0: fire(0)}

    lanes = lax.iota(jnp.int32, L)
    zero = jnp.zeros((L,), jnp.float32)
    UNROLL = 4

    sacc = [zero] * QN
    for j in range(NCH):
        if j + 1 < NCH:
            copies[j + 1] = fire(j + 1)
        for c in copies[j]:
            c.wait()
        bsel = j % 2

        def prod_body(i, carry, _j=j, _bsel=bsel):
            sq = list(carry)
            for u in range(UNROLL):
                b = i * UNROLL + u
                iv0 = in_buf[_bsel, b, pl.ds(0, L)]
                acc = iv0 * pos_buf[_bsel, b, pl.ds(0, L)]
                sq[0] += iv0
                for q in range(1, QN):
                    ivq = in_buf[_bsel, b, pl.ds(q * L, L)]
                    acc += ivq * pos_buf[_bsel, b, pl.ds(q * L, L)]
                    sq[q] += ivq
                plsc.store_scatter(part, [(_j * CH + b) * PAD + lanes], acc)
            return tuple(sq)

        sacc = list(lax.fori_loop(0, CH // UNROLL, prod_body, tuple(sacc)))

    for q in range(QN):
        sums_v[0, pl.ds(q * L, L)] = sacc[q]
        sums_v[1, pl.ds(q * L, L)] = zero

    @pl.when(sid == 0)
    def _():
        for j in range(NKCH):
            pltpu.sync_copy(neg.at[pl.ds(cid * KPC + j * CH, CH)], nidx.at[j])
        nc0 = pltpu.async_copy(out_tab.at[nidx.at[0]], neg_buf.at[0], nsem)
        nc1 = pltpu.async_copy(out_tab.at[nidx.at[1]], neg_buf.at[1], nsem)
        nc0.wait()
        nc1.wait()

        def neg_body(i, carry):
            nq = list(carry)
            for u in range(UNROLL):
                b = i * UNROLL + u
                for q in range(QN):
                    nq[q] += (neg_buf[0, b, pl.ds(q * L, L)] +
                              neg_buf[1, b, pl.ds(q * L, L)])
            return tuple(nq)

        nacc = lax.fori_loop(0, CH // UNROLL, neg_body, (zero,) * QN)
        for q in range(QN):
            sums_v[1, pl.ds(q * L, L)] = nacc[q]

    def red_body(t, carry):
        rowbase = (t * L + lanes) * PAD
        acc = plsc.load_gather(part, [rowbase])
        for jcol in range(1, L):
            acc += plsc.load_gather(part, [rowbase + jcol])
        scores_v[pl.ds(t * L, L)] = acc
        return carry

    lax.fori_loop(0, BPW // L, red_body, 0)

    pltpu.sync_copy(scores_v, scores_hbm.at[pl.ds(base, BPW)])
    pltpu.sync_copy(sums_v, sums_hbm.at[wid])


@functools.partial(
    pl.kernel,
    out_type=(
        jax.ShapeDtypeStruct((B,), jnp.float32),
        jax.ShapeDtypeStruct((NW, 2, D), jnp.float32),
    ),
    mesh=plsc.VectorSubcoreMesh(core_axis_name="c", subcore_axis_name="s"),
    compiler_params=pltpu.CompilerParams(
        needs_layout_passes=False, use_tc_tiling_on_sc=False),
    scratch_types=[
        pltpu.VMEM((NCH, CH), jnp.int32),
        pltpu.VMEM((NCH, CH), jnp.int32),
        pltpu.VMEM((NKCH, CH), jnp.int32),
        pltpu.VMEM((2, CH, DP), jnp.float32),
        pltpu.VMEM((2, CH, DP), jnp.float32),
        pltpu.VMEM((2, CH, DP), jnp.float32),
        pltpu.VMEM((BPW * PAD,), jnp.float32),
        pltpu.VMEM((BPW,), jnp.float32),
        pltpu.VMEM((2, D), jnp.float32),
        pltpu.SemaphoreType.DMA,
        pltpu.SemaphoreType.DMA,
        pltpu.SemaphoreType.DMA,
    ],
)
def _sc_stage(center, pos, neg, in_tab, out_tab, scores_hbm, sums_hbm,
              cidx, pidx, nidx, in_buf, pos_buf, neg_buf,
              part, scores_v, sums_v, sem, sem2, nsem):
    _sc_body(center, pos, neg, in_tab, out_tab, scores_hbm, sums_hbm,
             cidx, pidx, nidx, in_buf, pos_buf, neg_buf,
             part, scores_v, sums_v, sem, sem2, nsem)


def _tc_body(scores_ref, sums_ref, out_ref):
    s = scores_ref[...]
    total = jnp.sum(jax.nn.log_sigmoid(s))
    sm = sums_ref[...]
    c = jnp.sum(sm, axis=0, keepdims=True)
    ns = jnp.sum(c[:, 0:D] * c[:, D:2 * D])
    out_ref[...] = jnp.reshape(-(total + B * jax.nn.log_sigmoid(-ns)), (1, 1))


def kernel(center_word, positive_words, negative_words, input_table, output_table):
    in_tab = jnp.pad(input_table, ((0, 0), (0, DP - D)))
    out_tab = jnp.pad(output_table, ((0, 0), (0, DP - D)))
    scores, sums = _sc_stage(
        center_word.astype(jnp.int32),
        positive_words.astype(jnp.int32),
        negative_words.astype(jnp.int32),
        in_tab, out_tab)
    out = pl.pallas_call(
        _tc_body,
        out_shape=jax.ShapeDtypeStruct((1, 1), jnp.float32),
    )(scores.reshape(128, 128), sums.reshape(NW, 2 * D))
    return out[0, 0]


# speedup vs baseline: 1.4761x; 1.3395x over previous
import functools

import jax
import jax.numpy as jnp
from jax import lax
from jax.experimental import pallas as pl
from jax.experimental.pallas import tpu as pltpu
from jax.experimental.pallas import tpu_sc as plsc

V = 100000
D = 64
DP = 128
B = 16384
K = 512

NC = 2
NS = 16
NW = NC * NS
BPW = B // NW
CH = 128
NCH = BPW // CH
KPC = K // NC
NKCH = KPC // CH
PAD = 17
L = 16
QN = D // L


def _sc_body(center, pos, neg, in_tab, out_tab,
             scores_hbm, sums_hbm,
             cidx, pidx, nidx,
             in_buf, pos_buf, neg_buf,
             part, scores_v, sums_v,
             sem, sem2, nsem):
    cid = lax.axis_index("c")
    sid = lax.axis_index("s")
    wid = sid * NC + cid
    base = wid * BPW

    for j in range(NCH):
        pltpu.sync_copy(center.at[pl.ds(base + j * CH, CH)], cidx.at[j])
        pltpu.sync_copy(pos.at[pl.ds(base + j * CH, CH)], pidx.at[j])

    def fire(j):
        bsel = j % 2
        s = sem if bsel == 0 else sem2
        a = pltpu.async_copy(in_tab.at[cidx.at[j]], in_buf.at[bsel], s)
        b = pltpu.async_copy(out_tab.at[pidx.at[j]], pos_buf.at[bsel], s)
        return (a, b)

    copies = {0: fire(0)}

    lanes = lax.iota(jnp.int32, L)
    zero = jnp.zeros((L,), jnp.float32)
    UNROLL = 4

    sacc = [zero] * QN
    for j in range(NCH):
        if j + 1 < NCH:
            copies[j + 1] = fire(j + 1)
        for c in copies[j]:
            c.wait()
        bsel = j % 2

        def prod_body(i, carry, _j=j, _bsel=bsel):
            sq = list(carry)
            for u in range(UNROLL):
                b = i * UNROLL + u
                iv0 = in_buf[_bsel, b, pl.ds(0, L)]
                acc = iv0 * pos_buf[_bsel, b, pl.ds(0, L)]
                sq[0] += iv0
                for q in range(1, QN):
                    ivq = in_buf[_bsel, b, pl.ds(q * L, L)]
                    acc += ivq * pos_buf[_bsel, b, pl.ds(q * L, L)]
                    sq[q] += ivq
                plsc.store_scatter(part, [(_j * CH + b) * PAD + lanes], acc)
            return tuple(sq)

        sacc = list(lax.fori_loop(0, CH // UNROLL, prod_body, tuple(sacc)))

    for q in range(QN):
        sums_v[0, pl.ds(q * L, L)] = sacc[q]
        sums_v[1, pl.ds(q * L, L)] = zero

    @pl.when(sid == 0)
    def _():
        for j in range(NKCH):
            pltpu.sync_copy(neg.at[pl.ds(cid * KPC + j * CH, CH)], nidx.at[j])
        nc0 = pltpu.async_copy(out_tab.at[nidx.at[0]], neg_buf.at[0], nsem)
        nc1 = pltpu.async_copy(out_tab.at[nidx.at[1]], neg_buf.at[1], nsem)
        nc0.wait()
        nc1.wait()

        def neg_body(i, carry):
            nq = list(carry)
            for u in range(UNROLL):
                b = i * UNROLL + u
                for q in range(QN):
                    nq[q] += (neg_buf[0, b, pl.ds(q * L, L)] +
                              neg_buf[1, b, pl.ds(q * L, L)])
            return tuple(nq)

        nacc = lax.fori_loop(0, CH // UNROLL, neg_body, (zero,) * QN)
        for q in range(QN):
            sums_v[1, pl.ds(q * L, L)] = nacc[q]

    def red_body(t, carry):
        rowbase = (t * L + lanes) * PAD
        acc = plsc.load_gather(part, [rowbase])
        for jcol in range(1, L):
            acc += plsc.load_gather(part, [rowbase + jcol])
        scores_v[pl.ds(t * L, L)] = acc
        return carry

    lax.fori_loop(0, BPW // L, red_body, 0)

    pltpu.sync_copy(scores_v, scores_hbm.at[pl.ds(base, BPW)])
    pltpu.sync_copy(sums_v, sums_hbm.at[wid])


@functools.partial(
    pl.kernel,
    out_type=(
        jax.ShapeDtypeStruct((B,), jnp.float32),
        jax.ShapeDtypeStruct((NW, 2, D), jnp.float32),
    ),
    mesh=plsc.VectorSubcoreMesh(core_axis_name="c", subcore_axis_name="s"),
    compiler_params=pltpu.CompilerParams(
        needs_layout_passes=False, use_tc_tiling_on_sc=False),
    scratch_types=[
        pltpu.VMEM((NCH, CH), jnp.int32),
        pltpu.VMEM((NCH, CH), jnp.int32),
        pltpu.VMEM((NKCH, CH), jnp.int32),
        pltpu.VMEM((2, CH, DP), jnp.float32),
        pltpu.VMEM((2, CH, DP), jnp.float32),
        pltpu.VMEM((2, CH, DP), jnp.float32),
        pltpu.VMEM((BPW * PAD,), jnp.float32),
        pltpu.VMEM((BPW,), jnp.float32),
        pltpu.VMEM((2, D), jnp.float32),
        pltpu.SemaphoreType.DMA,
        pltpu.SemaphoreType.DMA,
        pltpu.SemaphoreType.DMA,
    ],
)
def _sc_stage(center, pos, neg, in_tab, out_tab, scores_hbm, sums_hbm,
              cidx, pidx, nidx, in_buf, pos_buf, neg_buf,
              part, scores_v, sums_v, sem, sem2, nsem):
    _sc_body(center, pos, neg, in_tab, out_tab, scores_hbm, sums_hbm,
             cidx, pidx, nidx, in_buf, pos_buf, neg_buf,
             part, scores_v, sums_v, sem, sem2, nsem)


TBLK = 2048


def _tr_body(in_t_ref, out_t_ref, o_in_ref, o_out_ref):
    a = in_t_ref[...]
    o_in_ref[...] = jnp.concatenate(
        [a.T, jnp.zeros((TBLK, DP - D), jnp.float32)], axis=1)
    b = out_t_ref[...]
    o_out_ref[...] = jnp.concatenate(
        [b.T, jnp.zeros((TBLK, DP - D), jnp.float32)], axis=1)


def _transpose_pad(in_t, out_t):
    nblk = (V + TBLK - 1) // TBLK
    return pl.pallas_call(
        _tr_body,
        grid=(nblk,),
        in_specs=[pl.BlockSpec((D, TBLK), lambda i: (0, i)),
                  pl.BlockSpec((D, TBLK), lambda i: (0, i))],
        out_specs=[pl.BlockSpec((TBLK, DP), lambda i: (i, 0)),
                   pl.BlockSpec((TBLK, DP), lambda i: (i, 0))],
        out_shape=[jax.ShapeDtypeStruct((V, DP), jnp.float32),
                   jax.ShapeDtypeStruct((V, DP), jnp.float32)],
    )(in_t, out_t)


def _tc_body(scores_ref, sums_ref, out_ref):
    s = scores_ref[...]
    total = jnp.sum(jax.nn.log_sigmoid(s))
    sm = sums_ref[...]
    c = jnp.sum(sm, axis=0, keepdims=True)
    ns = jnp.sum(c[:, 0:D] * c[:, D:2 * D])
    out_ref[...] = jnp.reshape(-(total + B * jax.nn.log_sigmoid(-ns)), (1, 1))


def kernel(center_word, positive_words, negative_words, input_table, output_table):
    in_tab, out_tab = _transpose_pad(input_table.T, output_table.T)
    scores, sums = _sc_stage(
        center_word.astype(jnp.int32),
        positive_words.astype(jnp.int32),
        negative_words.astype(jnp.int32),
        in_tab, out_tab)
    out = pl.pallas_call(
        _tc_body,
        out_shape=jax.ShapeDtypeStruct((1, 1), jnp.float32),
    )(scores.reshape(128, 128), sums.reshape(NW, 2 * D))
    return out[0, 0]


# speedup vs baseline: 1.7176x; 1.1637x over previous
import functools

import jax
import jax.numpy as jnp
from jax import lax
from jax.experimental import pallas as pl
from jax.experimental.pallas import tpu as pltpu
from jax.experimental.pallas import tpu_sc as plsc

V = 100000
D = 64
DP = 128
B = 16384
K = 512

NC = 2
NS = 16
NW = NC * NS
BPW = B // NW
CH = 128
NCH = BPW // CH
KPC = K // NC
NKCH = KPC // CH
PAD = 17
L = 16
QN = D // L
TBLK = 2048
VP = 51200


def _sc_body(center, pos, neg, in_tab, out_tab,
             scores_hbm, sums_hbm,
             cidx, pidx, nidx, cidx2, pidx2, nidx2,
             in_buf, pos_buf, neg_buf,
             part, scores_v, sums_v,
             sem, sem2, nsem):
    cid = lax.axis_index("c")
    sid = lax.axis_index("s")
    wid = sid * NC + cid
    base = wid * BPW

    for j in range(NCH):
        pltpu.sync_copy(center.at[pl.ds(base + j * CH, CH)], cidx.at[j])
        pltpu.sync_copy(pos.at[pl.ds(base + j * CH, CH)], pidx.at[j])

    half = VP
    for j in range(NCH):
        for q in range(CH // L):
            cv = cidx[j, pl.ds(q * L, L)]
            pv = pidx[j, pl.ds(q * L, L)]
            cidx2[j, pl.ds(q * L, L)] = jnp.where(cv >= half, cv - half, cv)
            pidx2[j, pl.ds(q * L, L)] = jnp.where(pv >= half, pv - half, pv)

    def fire(j):
        bsel = j % 2
        s = sem if bsel == 0 else sem2
        a = pltpu.async_copy(in_tab.at[cidx2.at[j]], in_buf.at[bsel], s)
        b = pltpu.async_copy(out_tab.at[pidx2.at[j]], pos_buf.at[bsel], s)
        return (a, b)

    copies = {0: fire(0)}

    lanes = lax.iota(jnp.int32, L)
    zero = jnp.zeros((L,), jnp.float32)
    UNROLL = 4

    sacc = [zero] * QN
    for j in range(NCH):
        if j + 1 < NCH:
            copies[j + 1] = fire(j + 1)
        for c in copies[j]:
            c.wait()
        bsel = j % 2

        def prod_body(i, carry, _j=j, _bsel=bsel):
            sq = list(carry)
            cvec = jnp.where(cidx[_j, pl.ds(i * L, L)] >= VP, D, 0)
            pvec = jnp.where(pidx[_j, pl.ds(i * L, L)] >= VP, D, 0)
            for u in range(L):
                b = i * L + u
                coff = cvec[u]
                poff = pvec[u]
                iv0 = in_buf[_bsel, b, pl.ds(coff, L)]
                acc = iv0 * pos_buf[_bsel, b, pl.ds(poff, L)]
                sq[0] += iv0
                for q in range(1, QN):
                    ivq = in_buf[_bsel, b, pl.ds(coff + q * L, L)]
                    acc += ivq * pos_buf[_bsel, b, pl.ds(poff + q * L, L)]
                    sq[q] += ivq
                plsc.store_scatter(part, [(_j * CH + b) * PAD + lanes], acc)
            return tuple(sq)

        sacc = list(lax.fori_loop(0, CH // L, prod_body, tuple(sacc)))

    for q in range(QN):
        sums_v[0, pl.ds(q * L, L)] = sacc[q]
        sums_v[1, pl.ds(q * L, L)] = zero

    @pl.when(sid == 0)
    def _():
        for j in range(NKCH):
            pltpu.sync_copy(neg.at[pl.ds(cid * KPC + j * CH, CH)], nidx.at[j])
        for j in range(NKCH):
            for q in range(CH // L):
                nv = nidx[j, pl.ds(q * L, L)]
                nidx2[j, pl.ds(q * L, L)] = jnp.where(
                    nv >= VP, nv - VP, nv)
        nc0 = pltpu.async_copy(out_tab.at[nidx2.at[0]], neg_buf.at[0], nsem)
        nc1 = pltpu.async_copy(out_tab.at[nidx2.at[1]], neg_buf.at[1], nsem)
        nc0.wait()
        nc1.wait()

        def neg_body(i, carry):
            nq = list(carry)
            nv0 = jnp.where(nidx[0, pl.ds(i * L, L)] >= VP, D, 0)
            nv1 = jnp.where(nidx[1, pl.ds(i * L, L)] >= VP, D, 0)
            for u in range(L):
                b = i * L + u
                noff0 = nv0[u]
                noff1 = nv1[u]
                for q in range(QN):
                    nq[q] += (neg_buf[0, b, pl.ds(noff0 + q * L, L)] +
                              neg_buf[1, b, pl.ds(noff1 + q * L, L)])
            return tuple(nq)

        nacc = lax.fori_loop(0, CH // L, neg_body, (zero,) * QN)
        for q in range(QN):
            sums_v[1, pl.ds(q * L, L)] = nacc[q]

    def red_body(t, carry):
        rowbase = (t * L + lanes) * PAD
        acc = plsc.load_gather(part, [rowbase])
        for jcol in range(1, L):
            acc += plsc.load_gather(part, [rowbase + jcol])
        scores_v[pl.ds(t * L, L)] = acc
        return carry

    lax.fori_loop(0, BPW // L, red_body, 0)

    pltpu.sync_copy(scores_v, scores_hbm.at[pl.ds(base, BPW)])
    pltpu.sync_copy(sums_v, sums_hbm.at[wid])


@functools.partial(
    pl.kernel,
    out_type=(
        jax.ShapeDtypeStruct((B,), jnp.float32),
        jax.ShapeDtypeStruct((NW, 2, D), jnp.float32),
    ),
    mesh=plsc.VectorSubcoreMesh(core_axis_name="c", subcore_axis_name="s"),
    compiler_params=pltpu.CompilerParams(
        needs_layout_passes=False, use_tc_tiling_on_sc=False),
    scratch_types=[
        pltpu.VMEM((NCH, CH), jnp.int32),
        pltpu.VMEM((NCH, CH), jnp.int32),
        pltpu.VMEM((NKCH, CH), jnp.int32),
        pltpu.VMEM((NCH, CH), jnp.int32),
        pltpu.VMEM((NCH, CH), jnp.int32),
        pltpu.VMEM((NKCH, CH), jnp.int32),
        pltpu.VMEM((2, CH, DP), jnp.float32),
        pltpu.VMEM((2, CH, DP), jnp.float32),
        pltpu.VMEM((2, CH, DP), jnp.float32),
        pltpu.VMEM((BPW * PAD,), jnp.float32),
        pltpu.VMEM((BPW,), jnp.float32),
        pltpu.VMEM((2, D), jnp.float32),
        pltpu.SemaphoreType.DMA,
        pltpu.SemaphoreType.DMA,
        pltpu.SemaphoreType.DMA,
    ],
)
def _sc_stage(center, pos, neg, in_tab, out_tab, scores_hbm, sums_hbm,
              cidx, pidx, nidx, cidx2, pidx2, nidx2, in_buf, pos_buf, neg_buf,
              part, scores_v, sums_v, sem, sem2, nsem):
    _sc_body(center, pos, neg, in_tab, out_tab, scores_hbm, sums_hbm,
             cidx, pidx, nidx, cidx2, pidx2, nidx2, in_buf, pos_buf, neg_buf,
             part, scores_v, sums_v, sem, sem2, nsem)


def _tr_body(in_lo, in_hi, out_lo, out_hi, o_in_ref, o_out_ref):
    o_in_ref[:, 0:D] = in_lo[...].T
    o_in_ref[:, D:DP] = in_hi[...].T
    o_out_ref[:, 0:D] = out_lo[...].T
    o_out_ref[:, D:DP] = out_hi[...].T


def _transpose_pack(in_t, out_t):
    nblk = VP // TBLK
    last = (V - 1) // TBLK
    lo = pl.BlockSpec((D, TBLK), lambda i: (0, i))
    hi = pl.BlockSpec((D, TBLK), lambda i: (0, jnp.minimum(i + VP // TBLK, last)))
    return pl.pallas_call(
        _tr_body,
        grid=(nblk,),
        in_specs=[lo, hi, lo, hi],
        out_specs=[pl.BlockSpec((TBLK, DP), lambda i: (i, 0)),
                   pl.BlockSpec((TBLK, DP), lambda i: (i, 0))],
        out_shape=[jax.ShapeDtypeStruct((VP, DP), jnp.float32),
                   jax.ShapeDtypeStruct((VP, DP), jnp.float32)],
    )(in_t, in_t, out_t, out_t)


def _tc_body(scores_ref, sums_ref, out_ref):
    s = scores_ref[...]
    total = jnp.sum(jax.nn.log_sigmoid(s))
    sm = sums_ref[...]
    c = jnp.sum(sm, axis=0, keepdims=True)
    ns = jnp.sum(c[:, 0:D] * c[:, D:2 * D])
    out_ref[...] = jnp.reshape(-(total + B * jax.nn.log_sigmoid(-ns)), (1, 1))


def kernel(center_word, positive_words, negative_words, input_table, output_table):
    in_tab, out_tab = _transpose_pack(input_table.T, output_table.T)
    scores, sums = _sc_stage(
        center_word.astype(jnp.int32),
        positive_words.astype(jnp.int32),
        negative_words.astype(jnp.int32),
        in_tab, out_tab)
    out = pl.pallas_call(
        _tc_body,
        out_shape=jax.ShapeDtypeStruct((1, 1), jnp.float32),
    )(scores.reshape(128, 128), sums.reshape(NW, 2 * D))
    return out[0, 0]
